# baseline (speedup 1.0000x reference)
import functools

import jax
import jax.numpy as jnp
from jax import lax
from jax.experimental import pallas as pl
from jax.experimental.pallas import tpu as pltpu
from jax.experimental.pallas import tpu_sc as plsc

NC = 2
NS = 16
NW = NC * NS
CH = 128
RPT = 640
NPAD = NS * RPT
BN = 2048
FRAC0 = 0.460


def _cdiv(a, b):
    return (a + b - 1) // b


def _split8(total, nt):
    eights = total // 8
    return eights // nt, eights % nt, total % 8


def _mesh():
    return plsc.VectorSubcoreMesh(
        core_axis_name="c", subcore_axis_name="s", num_cores=NC, num_subcores=NS
    )


def _ranged_load(tid, q, r8, tail, nt, load_fn):
    last = nt - 1
    v_hi = 8 * (q + 1)
    v_lo = 8 * q
    v_last = 8 * (q + (1 if last < r8 else 0)) + tail
    if tail:
        if v_last:
            @pl.when(tid == last)
            def _():
                load_fn(v_last)
        if v_hi:
            @pl.when(jnp.logical_and(tid != last, tid < r8))
            def _():
                load_fn(v_hi)
        if v_lo:
            @pl.when(jnp.logical_and(tid != last, tid >= r8))
            def _():
                load_fn(v_lo)
    else:
        if v_hi and r8:
            @pl.when(tid < r8)
            def _():
                load_fn(v_hi)
        if v_lo:
            @pl.when(tid >= r8)
            def _():
                load_fn(v_lo)


def _ranged_params(tid, q, r8, tail, nt):
    base = 8 * (q * tid + jnp.minimum(tid, r8))
    base = pl.multiple_of(base, 8)
    nch = 8 * q + jnp.where(tid < r8, 8, 0)
    if tail:
        nch = nch + jnp.where(tid == nt - 1, tail, 0)
    return base, nch


@functools.lru_cache(maxsize=None)
def _make_deg(total_ch):
    q, r8, tail = _split8(total_ch, NW)
    nch_max = 8 * (q + (1 if r8 else 0)) + tail

    @functools.partial(
        pl.kernel,
        out_type=jax.ShapeDtypeStruct((NW, NPAD), jnp.float32),
        mesh=_mesh(),
        scratch_types=[
            pltpu.VMEM((nch_max, CH), jnp.int32),
            pltpu.VMEM((NPAD,), jnp.float32),
        ],
        compiler_params=pltpu.CompilerParams(needs_layout_passes=False),
    )
    def deg_kernel(edges_hbm, zeros_hbm, out_hbm, didx, acc):
        c = lax.axis_index("c")
        s = lax.axis_index("s")
        wid = s * NC + c
        base, nch = _ranged_params(wid, q, r8, tail, NW)

        def load_fn(v):
            pltpu.sync_copy(edges_hbm.at[1, pl.ds(base, v)],
                            didx.at[pl.ds(0, v)])

        _ranged_load(wid, q, r8, tail, NW, load_fn)
        pltpu.sync_copy(zeros_hbm, acc)
        ones16 = jnp.ones((16,), jnp.float32)

        def body(g, carry):
            for j in range(CH // 16):
                idx = didx[g, pl.ds(j * 16, 16)]
                plsc.addupdate_scatter(acc, [idx], ones16)
            return carry

        lax.fori_loop(0, nch, body, 0)
        pltpu.sync_copy(acc, out_hbm.at[wid])

    return deg_kernel


@functools.lru_cache(maxsize=None)
def _make_agg(total_ch, width):
    a0 = min(int(round(FRAC0 * total_ch / NS / 8)) * 8, (total_ch // NS) // 8 * 8)
    rest = total_ch - NS * a0
    q1, r81, tail1 = _split8(rest, NS)
    core1_base = NS * a0
    nch_max = max(a0, 8 * (q1 + (1 if r81 else 0)) + tail1)

    @functools.partial(
        pl.kernel,
        out_type=jax.ShapeDtypeStruct((NC, NPAD, width), jnp.float32),
        mesh=_mesh(),
        scratch_types=[
            pltpu.VMEM((nch_max, CH), jnp.int32),
            pltpu.VMEM((nch_max, CH), jnp.int32),
            pltpu.VMEM((CH, width), jnp.float32),
            pltpu.VMEM_SHARED((NPAD, width), jnp.float32),
            pltpu.SemaphoreType.DMA,
        ],
    )
    def agg_kernel(feat_hbm, edges_hbm, zeros_hbm, out_hbm,
                   sidx, didx, rows, acc, sem):
        c = lax.axis_index("c")
        s = lax.axis_index("s")
        base1, nch1 = _ranged_params(s, q1, r81, tail1, NS)
        base = jnp.where(c == 0, s * a0, core1_base + base1)
        base = pl.multiple_of(base, 8)
        nch = jnp.where(c == 0, a0, nch1)

        def load_idx(n):
            pltpu.sync_copy(edges_hbm.at[0, pl.ds(base, n)],
                            sidx.at[pl.ds(0, n)])
            pltpu.sync_copy(edges_hbm.at[1, pl.ds(base, n)],
                            didx.at[pl.ds(0, n)])

        if a0:
            @pl.when(c == 0)
            def _():
                load_idx(a0)

        @pl.when(c == 1)
        def _():
            _ranged_load(s, q1, r81, tail1, NS, load_idx)

        pltpu.sync_copy(zeros_hbm, acc.at[pl.ds(s * RPT, RPT)])
        plsc.subcore_barrier()

        def body(g, carry):
            pltpu.async_copy(feat_hbm.at[sidx.at[g]], rows, sem).wait()
            pltpu.sync_copy(rows, acc.at[didx.at[g]], add=True)
            return carry

        lax.fori_loop(0, nch, body, 0)
        plsc.subcore_barrier()
        pltpu.sync_copy(
            acc.at[pl.ds(s * RPT, RPT)], out_hbm.at[c, pl.ds(s * RPT, RPT)]
        )

    return agg_kernel


def _dinv_from_partials(deg_ref):
    deg = jnp.sum(deg_ref[...], axis=0)
    return lax.rsqrt(jnp.maximum(deg, 1.0))


def _tc1_body(x_ref, w_ref, deg_ref, o_ref):
    dinv = _dinv_from_partials(deg_ref)
    h = jnp.dot(x_ref[...], w_ref[...], preferred_element_type=jnp.float32)
    o_ref[...] = h * dinv[:, None]


def _tc2_body(agg_ref, deg_ref, o_ref):
    dinv = _dinv_from_partials(deg_ref)
    raw = agg_ref[0] + agg_ref[1]
    h1 = jnp.maximum(raw * dinv[:, None], 0.0)
    o_ref[...] = h1 * dinv[:, None]


def _tc3_body(agg_ref, deg_ref, w_ref, o_ref):
    dinv = _dinv_from_partials(deg_ref)
    scaled = (agg_ref[0] + agg_ref[1]) * dinv[:, None]
    o_ref[...] = jnp.dot(scaled, w_ref[...], preferred_element_type=jnp.float32)


@functools.lru_cache(maxsize=None)
def _make_tc1(d, h):
    grid = (NPAD // BN,)
    return pl.pallas_call(
        _tc1_body,
        grid=grid,
        in_specs=[
            pl.BlockSpec((BN, d), lambda i: (i, 0)),
            pl.BlockSpec((d, h), lambda i: (0, 0)),
            pl.BlockSpec((NW, BN), lambda i: (0, i)),
        ],
        out_specs=pl.BlockSpec((BN, h), lambda i: (i, 0)),
        out_shape=jax.ShapeDtypeStruct((NPAD, h), jnp.float32),
    )


@functools.lru_cache(maxsize=None)
def _make_tc2(h):
    grid = (NPAD // BN,)
    return pl.pallas_call(
        _tc2_body,
        grid=grid,
        in_specs=[
            pl.BlockSpec((NC, BN, h), lambda i: (0, i, 0)),
            pl.BlockSpec((NW, BN), lambda i: (0, i)),
        ],
        out_specs=pl.BlockSpec((BN, h), lambda i: (i, 0)),
        out_shape=jax.ShapeDtypeStruct((NPAD, h), jnp.float32),
    )


@functools.lru_cache(maxsize=None)
def _make_tc3(h, cdim):
    grid = (NPAD // BN,)
    return pl.pallas_call(
        _tc3_body,
        grid=grid,
        in_specs=[
            pl.BlockSpec((NC, BN, h), lambda i: (0, i, 0)),
            pl.BlockSpec((NW, BN), lambda i: (0, i)),
            pl.BlockSpec((h, cdim), lambda i: (0, 0)),
        ],
        out_specs=pl.BlockSpec((BN, cdim), lambda i: (i, 0)),
        out_shape=jax.ShapeDtypeStruct((NPAD, cdim), jnp.float32),
    )


def kernel(x, edge_index, W1, W2):
    n, d = x.shape
    h = W1.shape[1]
    cdim = W2.shape[1]
    e = edge_index.shape[1]

    if e % (8 * CH):
        pad_e = 8 * CH - e % (8 * CH)
        edge_index = jnp.concatenate(
            [
                edge_index,
                jnp.stack(
                    [
                        jnp.zeros((pad_e,), jnp.int32),
                        jnp.full((pad_e,), n, jnp.int32),
                    ]
                ),
            ],
            axis=1,
        )
    total_ch = edge_index.shape[1] // CH
    edges3 = edge_index.reshape(2, total_ch, CH)

    x_p = jnp.pad(x, ((0, NPAD - n), (0, 0)))
    zeros_deg = jnp.zeros((NPAD,), jnp.float32)
    zeros_h = jnp.zeros((RPT, h), jnp.float32)

    degp = _make_deg(total_ch)(edges3, zeros_deg)
    hp = _make_tc1(d, h)(x_p, W1, degp)
    agg = _make_agg(total_ch, h)
    aggp = agg(hp, edges3, zeros_h)
    h1p = _make_tc2(h)(aggp, degp)
    outp = agg(h1p, edges3, zeros_h)
    out_full = _make_tc3(h, cdim)(outp, degp, W2)
    return out_full[:n]


# speedup vs baseline: 1.0731x; 1.0731x over previous
import functools

import jax
import jax.numpy as jnp
from jax import lax
from jax.experimental import pallas as pl
from jax.experimental.pallas import tpu as pltpu
from jax.experimental.pallas import tpu_sc as plsc

NC = 2
NS = 16
NW = NC * NS
CH = 128
RPT = 640
NPAD = NS * RPT
BN = 2048
FRAC0 = 0.511


def _cdiv(a, b):
    return (a + b - 1) // b


def _split8(total, nt):
    eights = total // 8
    return eights // nt, eights % nt, total % 8


def _mesh():
    return plsc.VectorSubcoreMesh(
        core_axis_name="c", subcore_axis_name="s", num_cores=NC, num_subcores=NS
    )


def _ranged_load(tid, q, r8, tail, nt, load_fn):
    last = nt - 1
    v_hi = 8 * (q + 1)
    v_lo = 8 * q
    v_last = 8 * (q + (1 if last < r8 else 0)) + tail
    if tail:
        if v_last:
            @pl.when(tid == last)
            def _():
                load_fn(v_last)
        if v_hi:
            @pl.when(jnp.logical_and(tid != last, tid < r8))
            def _():
                load_fn(v_hi)
        if v_lo:
            @pl.when(jnp.logical_and(tid != last, tid >= r8))
            def _():
                load_fn(v_lo)
    else:
        if v_hi and r8:
            @pl.when(tid < r8)
            def _():
                load_fn(v_hi)
        if v_lo:
            @pl.when(tid >= r8)
            def _():
                load_fn(v_lo)


def _ranged_params(tid, q, r8, tail, nt):
    base = 8 * (q * tid + jnp.minimum(tid, r8))
    base = pl.multiple_of(base, 8)
    nch = 8 * q + jnp.where(tid < r8, 8, 0)
    if tail:
        nch = nch + jnp.where(tid == nt - 1, tail, 0)
    return base, nch


@functools.lru_cache(maxsize=None)
def _make_deg(total_ch):
    q, r8, tail = _split8(total_ch, NW)
    nch_max = 8 * (q + (1 if r8 else 0)) + tail

    @functools.partial(
        pl.kernel,
        out_type=jax.ShapeDtypeStruct((NW, NPAD), jnp.float32),
        mesh=_mesh(),
        scratch_types=[
            pltpu.VMEM((nch_max, CH), jnp.int32),
            pltpu.VMEM((NPAD,), jnp.float32),
        ],
        compiler_params=pltpu.CompilerParams(needs_layout_passes=False),
    )
    def deg_kernel(edges_hbm, zeros_hbm, out_hbm, didx, acc):
        c = lax.axis_index("c")
        s = lax.axis_index("s")
        wid = s * NC + c
        base, nch = _ranged_params(wid, q, r8, tail, NW)

        def load_fn(v):
            pltpu.sync_copy(edges_hbm.at[1, pl.ds(base, v)],
                            didx.at[pl.ds(0, v)])

        _ranged_load(wid, q, r8, tail, NW, load_fn)
        pltpu.sync_copy(zeros_hbm, acc)
        ones16 = jnp.ones((16,), jnp.float32)

        def body(g, carry):
            for j in range(CH // 16):
                idx = didx[g, pl.ds(j * 16, 16)]
                plsc.addupdate_scatter(acc, [idx], ones16)
            return carry

        lax.fori_loop(0, nch, body, 0)
        pltpu.sync_copy(acc, out_hbm.at[wid])

    return deg_kernel


@functools.lru_cache(maxsize=None)
def _make_agg(total_ch, width):
    a0 = min(int(round(FRAC0 * total_ch / NS / 8)) * 8, (total_ch // NS) // 8 * 8)
    rest = total_ch - NS * a0
    q1, r81, tail1 = _split8(rest, NS)
    core1_base = NS * a0
    nch_max = max(a0, 8 * (q1 + (1 if r81 else 0)) + tail1)

    @functools.partial(
        pl.kernel,
        out_type=jax.ShapeDtypeStruct((NC, NPAD, width), jnp.float32),
        mesh=_mesh(),
        scratch_types=[
            pltpu.VMEM((nch_max, CH), jnp.int32),
            pltpu.VMEM((nch_max, CH), jnp.int32),
            pltpu.VMEM((CH, width), jnp.float32),
            pltpu.VMEM_SHARED((NPAD, width), jnp.float32),
            pltpu.SemaphoreType.DMA,
        ],
    )
    def agg_kernel(feat_hbm, edges_hbm, zeros_hbm, out_hbm,
                   sidx, didx, rows, acc, sem):
        c = lax.axis_index("c")
        s = lax.axis_index("s")
        base1, nch1 = _ranged_params(s, q1, r81, tail1, NS)
        base = jnp.where(c == 0, s * a0, core1_base + base1)
        base = pl.multiple_of(base, 8)
        nch = jnp.where(c == 0, a0, nch1)

        def load_idx(n):
            pltpu.sync_copy(edges_hbm.at[0, pl.ds(base, n)],
                            sidx.at[pl.ds(0, n)])
            pltpu.sync_copy(edges_hbm.at[1, pl.ds(base, n)],
                            didx.at[pl.ds(0, n)])

        if a0:
            @pl.when(c == 0)
            def _():
                load_idx(a0)

        @pl.when(c == 1)
        def _():
            _ranged_load(s, q1, r81, tail1, NS, load_idx)

        pltpu.sync_copy(zeros_hbm, acc.at[pl.ds(s * RPT, RPT)])
        plsc.subcore_barrier()

        def body(g, carry):
            pltpu.async_copy(feat_hbm.at[sidx.at[g]], rows, sem).wait()
            pltpu.sync_copy(rows, acc.at[didx.at[g]], add=True)
            return carry

        lax.fori_loop(0, nch, body, 0)
        plsc.subcore_barrier()
        pltpu.sync_copy(
            acc.at[pl.ds(s * RPT, RPT)], out_hbm.at[c, pl.ds(s * RPT, RPT)]
        )

    return agg_kernel


def _dinv_from_partials(deg_ref):
    deg = jnp.sum(deg_ref[...], axis=0)
    return lax.rsqrt(jnp.maximum(deg, 1.0))


def _tc1_body(x_ref, w_ref, deg_ref, o_ref):
    dinv = _dinv_from_partials(deg_ref)
    h = jnp.dot(x_ref[...], w_ref[...], preferred_element_type=jnp.float32)
    o_ref[...] = h * dinv[:, None]


def _tc2_body(agg_ref, deg_ref, o_ref):
    dinv = _dinv_from_partials(deg_ref)
    raw = agg_ref[0] + agg_ref[1]
    h1 = jnp.maximum(raw * dinv[:, None], 0.0)
    o_ref[...] = h1 * dinv[:, None]


def _tc3_body(agg_ref, deg_ref, w_ref, o_ref):
    dinv = _dinv_from_partials(deg_ref)
    scaled = (agg_ref[0] + agg_ref[1]) * dinv[:, None]
    o_ref[...] = jnp.dot(scaled, w_ref[...], preferred_element_type=jnp.float32)


@functools.lru_cache(maxsize=None)
def _make_tc1(d, h):
    grid = (NPAD // BN,)
    return pl.pallas_call(
        _tc1_body,
        grid=grid,
        in_specs=[
            pl.BlockSpec((BN, d), lambda i: (i, 0)),
            pl.BlockSpec((d, h), lambda i: (0, 0)),
            pl.BlockSpec((NW, BN), lambda i: (0, i)),
        ],
        out_specs=pl.BlockSpec((BN, h), lambda i: (i, 0)),
        out_shape=jax.ShapeDtypeStruct((NPAD, h), jnp.float32),
    )


@functools.lru_cache(maxsize=None)
def _make_tc2(h):
    grid = (NPAD // BN,)
    return pl.pallas_call(
        _tc2_body,
        grid=grid,
        in_specs=[
            pl.BlockSpec((NC, BN, h), lambda i: (0, i, 0)),
            pl.BlockSpec((NW, BN), lambda i: (0, i)),
        ],
        out_specs=pl.BlockSpec((BN, h), lambda i: (i, 0)),
        out_shape=jax.ShapeDtypeStruct((NPAD, h), jnp.float32),
    )


@functools.lru_cache(maxsize=None)
def _make_tc3(h, cdim):
    grid = (NPAD // BN,)
    return pl.pallas_call(
        _tc3_body,
        grid=grid,
        in_specs=[
            pl.BlockSpec((NC, BN, h), lambda i: (0, i, 0)),
            pl.BlockSpec((NW, BN), lambda i: (0, i)),
            pl.BlockSpec((h, cdim), lambda i: (0, 0)),
        ],
        out_specs=pl.BlockSpec((BN, cdim), lambda i: (i, 0)),
        out_shape=jax.ShapeDtypeStruct((NPAD, cdim), jnp.float32),
    )


def kernel(x, edge_index, W1, W2):
    n, d = x.shape
    h = W1.shape[1]
    cdim = W2.shape[1]
    e = edge_index.shape[1]

    if e % (8 * CH):
        pad_e = 8 * CH - e % (8 * CH)
        edge_index = jnp.concatenate(
            [
                edge_index,
                jnp.stack(
                    [
                        jnp.zeros((pad_e,), jnp.int32),
                        jnp.full((pad_e,), n, jnp.int32),
                    ]
                ),
            ],
            axis=1,
        )
    total_ch = edge_index.shape[1] // CH
    edges3 = edge_index.reshape(2, total_ch, CH)

    x_p = jnp.pad(x, ((0, NPAD - n), (0, 0)))
    zeros_deg = jnp.zeros((NPAD,), jnp.float32)
    zeros_h = jnp.zeros((RPT, h), jnp.float32)

    degp = _make_deg(total_ch)(edges3, zeros_deg)
    hp = _make_tc1(d, h)(x_p, W1, degp)
    agg = _make_agg(total_ch, h)
    aggp = agg(hp, edges3, zeros_h)
    h1p = _make_tc2(h)(aggp, degp)
    outp = agg(h1p, edges3, zeros_h)
    out_full = _make_tc3(h, cdim)(outp, degp, W2)
    return out_full[:n]


# speedup vs baseline: 1.4398x; 1.3417x over previous
import functools

import jax
import jax.numpy as jnp
from jax import lax
from jax.experimental import pallas as pl
from jax.experimental.pallas import tpu as pltpu
from jax.experimental.pallas import tpu_sc as plsc

NC = 2
NS = 16
NW = NC * NS
CH = 128
RPT = 640
NPAD = NS * RPT
BN = 2048
FRAC0 = 0.511


def _cdiv(a, b):
    return (a + b - 1) // b


def _split8(total, nt):
    eights = total // 8
    return eights // nt, eights % nt, total % 8


def _mesh():
    return plsc.VectorSubcoreMesh(
        core_axis_name="c", subcore_axis_name="s", num_cores=NC, num_subcores=NS
    )


def _ranged_load(tid, q, r8, tail, nt, load_fn):
    last = nt - 1
    v_hi = 8 * (q + 1)
    v_lo = 8 * q
    v_last = 8 * (q + (1 if last < r8 else 0)) + tail
    if tail:
        if v_last:
            @pl.when(tid == last)
            def _():
                load_fn(v_last)
        if v_hi:
            @pl.when(jnp.logical_and(tid != last, tid < r8))
            def _():
                load_fn(v_hi)
        if v_lo:
            @pl.when(jnp.logical_and(tid != last, tid >= r8))
            def _():
                load_fn(v_lo)
    else:
        if v_hi and r8:
            @pl.when(tid < r8)
            def _():
                load_fn(v_hi)
        if v_lo:
            @pl.when(tid >= r8)
            def _():
                load_fn(v_lo)


def _ranged_params(tid, q, r8, tail, nt):
    base = 8 * (q * tid + jnp.minimum(tid, r8))
    base = pl.multiple_of(base, 8)
    nch = 8 * q + jnp.where(tid < r8, 8, 0)
    if tail:
        nch = nch + jnp.where(tid == nt - 1, tail, 0)
    return base, nch


@functools.lru_cache(maxsize=None)
def _make_deg(total_ch):
    q, r8, tail = _split8(total_ch, NW)
    nch_max = 8 * (q + (1 if r8 else 0)) + tail

    @functools.partial(
        pl.kernel,
        out_type=jax.ShapeDtypeStruct((NW, NPAD), jnp.float32),
        mesh=_mesh(),
        scratch_types=[
            pltpu.VMEM((nch_max, CH), jnp.int32),
            pltpu.VMEM((NPAD,), jnp.float32),
        ],
        compiler_params=pltpu.CompilerParams(needs_layout_passes=False),
    )
    def deg_kernel(edges_hbm, zeros_hbm, out_hbm, didx, acc):
        c = lax.axis_index("c")
        s = lax.axis_index("s")
        wid = s * NC + c
        base, nch = _ranged_params(wid, q, r8, tail, NW)

        def load_fn(v):
            pltpu.sync_copy(edges_hbm.at[1, pl.ds(base, v)],
                            didx.at[pl.ds(0, v)])

        _ranged_load(wid, q, r8, tail, NW, load_fn)
        pltpu.sync_copy(zeros_hbm, acc)
        ones16 = jnp.ones((16,), jnp.float32)

        def body(g, carry):
            for j in range(CH // 16):
                idx = didx[g, pl.ds(j * 16, 16)]
                plsc.addupdate_scatter(acc, [idx], ones16)
            return carry

        lax.fori_loop(0, nch, body, 0)
        pltpu.sync_copy(acc, out_hbm.at[wid])

    return deg_kernel


def _agg_split(total_ch):
    a0 = min(int(round(FRAC0 * total_ch / NS / 8)) * 8, (total_ch // NS) // 8 * 8)
    rest = total_ch - NS * a0
    q1, r81, tail1 = _split8(rest, NS)
    nch_max = max(a0, 8 * (q1 + (1 if r81 else 0)) + tail1)
    half_max = _cdiv(_cdiv(nch_max, 2), 8) * 8
    return a0, q1, r81, tail1, nch_max, half_max


@functools.lru_cache(maxsize=None)
def _make_agg(total_ch, width):
    a0, q1, r81, tail1, nch_max, half_max = _agg_split(total_ch)
    core1_base = NS * a0

    @functools.partial(
        pl.kernel,
        out_type=jax.ShapeDtypeStruct((NC, NPAD, width), jnp.float32),
        mesh=_mesh(),
        scratch_types=[
            pltpu.VMEM((half_max, CH), jnp.int32),
            pltpu.VMEM((half_max, CH), jnp.int32),
            pltpu.VMEM((2, CH, width), jnp.float32),
            pltpu.VMEM_SHARED((NPAD, width), jnp.float32),
            pltpu.SemaphoreType.DMA((2,)),
            pltpu.SemaphoreType.DMA((2,)),
        ],
    )
    def agg_kernel(feat_hbm, edges_hbm, zeros_hbm, out_hbm,
                   sidx, didx, rows, acc, gsem, ssem):
        c = lax.axis_index("c")
        s = lax.axis_index("s")
        base1, nch1 = _ranged_params(s, q1, r81, tail1, NS)
        base = jnp.where(c == 0, s * a0, core1_base + base1)
        base = pl.multiple_of(base, 8)
        nch = jnp.where(c == 0, a0, nch1)

        pltpu.sync_copy(zeros_hbm, acc.at[pl.ds(s * RPT, RPT)])
        plsc.subcore_barrier()

        def gather(g, b):
            pltpu.async_copy(feat_hbm.at[sidx.at[g]], rows.at[b], gsem.at[b])

        def gather_wait(g, b):
            pltpu.make_async_copy(
                feat_hbm.at[sidx.at[g]], rows.at[b], gsem.at[b]
            ).wait()

        def scatter(g, b):
            pltpu.async_copy(rows.at[b], acc.at[didx.at[g]], ssem.at[b],
                             add=True)

        def scatter_wait(g, b):
            pltpu.make_async_copy(
                rows.at[b], acc.at[didx.at[g]], ssem.at[b]
            ).wait()

        def load_half(hbase, n):
            pltpu.sync_copy(edges_hbm.at[0, pl.ds(hbase, n)],
                            sidx.at[pl.ds(0, n)])
            pltpu.sync_copy(edges_hbm.at[1, pl.ds(hbase, n)],
                            didx.at[pl.ds(0, n)])

        def run_half(cnt):
            gather(0, 0)
            gather(1, 1)

            def body(g, carry):
                b = g % 2
                gather_wait(g, b)
                scatter(g, b)
                scatter_wait(g, b)
                gather(g + 2, b)
                return carry

            lax.fori_loop(0, cnt - 2, body, 0)

            def tail(g, carry):
                b = g % 2
                gather_wait(g, b)
                scatter(g, b)
                scatter_wait(g, b)
                return carry

            lax.fori_loop(cnt - 2, cnt, tail, 0)

        h1 = (nch // 16) * 8
        h1 = pl.multiple_of(h1, 8)
        h2 = nch - h1

        def stage_and_run(hbase, n_static, cnt):
            @pl.when(cnt > 0)
            def _():
                load_half(hbase, n_static)
                run_half(cnt)

        stage_and_run(base, half_max, h1)
        stage_and_run(base + h1, half_max, h2)

        plsc.subcore_barrier()
        pltpu.sync_copy(
            acc.at[pl.ds(s * RPT, RPT)], out_hbm.at[c, pl.ds(s * RPT, RPT)]
        )

    return agg_kernel


def _dinv_from_partials(deg_ref):
    deg = jnp.sum(deg_ref[...], axis=0)
    return lax.rsqrt(jnp.maximum(deg, 1.0))


def _tc1_body(x_ref, w_ref, deg_ref, o_ref):
    dinv = _dinv_from_partials(deg_ref)
    h = jnp.dot(x_ref[...], w_ref[...], preferred_element_type=jnp.float32)
    o_ref[...] = h * dinv[:, None]


def _tc2_body(agg_ref, deg_ref, o_ref):
    dinv = _dinv_from_partials(deg_ref)
    raw = agg_ref[0] + agg_ref[1]
    h1 = jnp.maximum(raw * dinv[:, None], 0.0)
    o_ref[...] = h1 * dinv[:, None]


def _tc3_body(agg_ref, deg_ref, w_ref, o_ref):
    dinv = _dinv_from_partials(deg_ref)
    scaled = (agg_ref[0] + agg_ref[1]) * dinv[:, None]
    o_ref[...] = jnp.dot(scaled, w_ref[...], preferred_element_type=jnp.float32)


@functools.lru_cache(maxsize=None)
def _make_tc1(d, h):
    grid = (NPAD // BN,)
    return pl.pallas_call(
        _tc1_body,
        grid=grid,
        in_specs=[
            pl.BlockSpec((BN, d), lambda i: (i, 0)),
            pl.BlockSpec((d, h), lambda i: (0, 0)),
            pl.BlockSpec((NW, BN), lambda i: (0, i)),
        ],
        out_specs=pl.BlockSpec((BN, h), lambda i: (i, 0)),
        out_shape=jax.ShapeDtypeStruct((NPAD, h), jnp.float32),
    )


@functools.lru_cache(maxsize=None)
def _make_tc2(h):
    grid = (NPAD // BN,)
    return pl.pallas_call(
        _tc2_body,
        grid=grid,
        in_specs=[
            pl.BlockSpec((NC, BN, h), lambda i: (0, i, 0)),
            pl.BlockSpec((NW, BN), lambda i: (0, i)),
        ],
        out_specs=pl.BlockSpec((BN, h), lambda i: (i, 0)),
        out_shape=jax.ShapeDtypeStruct((NPAD, h), jnp.float32),
    )


@functools.lru_cache(maxsize=None)
def _make_tc3(h, cdim):
    grid = (NPAD // BN,)
    return pl.pallas_call(
        _tc3_body,
        grid=grid,
        in_specs=[
            pl.BlockSpec((NC, BN, h), lambda i: (0, i, 0)),
            pl.BlockSpec((NW, BN), lambda i: (0, i)),
            pl.BlockSpec((h, cdim), lambda i: (0, 0)),
        ],
        out_specs=pl.BlockSpec((BN, cdim), lambda i: (i, 0)),
        out_shape=jax.ShapeDtypeStruct((NPAD, cdim), jnp.float32),
    )


def kernel(x, edge_index, W1, W2):
    n, d = x.shape
    h = W1.shape[1]
    cdim = W2.shape[1]
    e = edge_index.shape[1]

    if e % (8 * CH):
        pad_e = 8 * CH - e % (8 * CH)
        edge_index = jnp.concatenate(
            [
                edge_index,
                jnp.stack(
                    [
                        jnp.zeros((pad_e,), jnp.int32),
                        jnp.full((pad_e,), n, jnp.int32),
                    ]
                ),
            ],
            axis=1,
        )
    total_ch = edge_index.shape[1] // CH
    edges3 = edge_index.reshape(2, total_ch, CH)
    half_max = _agg_split(total_ch)[5]
    edges3 = jnp.pad(edges3, ((0, 0), (0, half_max), (0, 0)))

    x_p = jnp.pad(x, ((0, NPAD - n), (0, 0)))
    zeros_deg = jnp.zeros((NPAD,), jnp.float32)
    zeros_h = jnp.zeros((RPT, h), jnp.float32)

    degp = _make_deg(total_ch)(edges3, zeros_deg)
    hp = _make_tc1(d, h)(x_p, W1, degp)
    agg = _make_agg(total_ch, h)
    aggp = agg(hp, edges3, zeros_h)
    h1p = _make_tc2(h)(aggp, degp)
    outp = agg(h1p, edges3, zeros_h)
    out_full = _make_tc3(h, cdim)(outp, degp, W2)
    return out_full[:n]
